# baseline (speedup 1.0000x reference)
import functools

import jax
import jax.numpy as jnp
from jax.experimental import pallas as pl
from jax.experimental.pallas import tpu as pltpu

K = 5
GAMMA = 1.0
NEG_INF = float("-inf")


def _prep_kernel(xf_ref, xz_ref, ps_ref, a_ref, as_ref, msim_ref, pidx_ref,
                 *, nf):
    xf = xf_ref[...]
    xz = xz_ref[...]
    nrm_f = jnp.maximum(jnp.sqrt(jnp.sum(xf * xf, axis=1, keepdims=True)), 1e-8)
    nrm_z = jnp.maximum(jnp.sqrt(jnp.sum(xz * xz, axis=1, keepdims=True)), 1e-8)
    xnf = xf / nrm_f
    xnz = xz / nrm_z
    a_ref[:, :nf] = xnf
    a_ref[:, nf:] = xnz
    as_ref[:, :nf] = xnf
    as_ref[:, nf:] = -xnz

    ps = ps_ref[...]
    pp = ps.shape[1]
    pmax = jnp.max(ps, axis=1, keepdims=True)
    li = jax.lax.broadcasted_iota(jnp.int32, ps.shape, 1)
    pidx = jnp.min(jnp.where(ps == pmax, li, pp), axis=1, keepdims=True)
    msim_ref[...] = pmax
    pidx_ref[...] = pidx


def _search_kernel(a_rows_ref, as_full_ref, msim_r_ref, pidx_r_ref, ext_r_ref,
                   msim_c_ref, pidx_c_ref, ext_c_ref, out_ref,
                   rvals, rflats, rowscratch, *, bn, nf, tr, ngrid):
    i = pl.program_id(0)
    j = pl.program_id(1)

    @pl.when((i == 0) & (j == 0))
    def _init():
        for k in range(K):
            rvals[k] = jnp.float32(NEG_INF)
            rflats[k] = jnp.int32(-1)

    @pl.when(j >= i)
    def _cell():
        v = jax.lax.dot_general(
            a_rows_ref[...], as_full_ref[pl.ds(j * tr, tr), :],
            dimension_numbers=(((1,), (1,)), ((), ())),
            preferred_element_type=jnp.float32,
        )
        ir = jax.lax.broadcasted_iota(jnp.int32, (tr, 1), 0) + i * tr
        ic = jax.lax.broadcasted_iota(jnp.int32, (1, tr), 1) + j * tr
        inv = (jnp.where(ir < bn, 0.0, NEG_INF)
               + jnp.where(ic < bn, 0.0, NEG_INF))
        cand = (
            (ir < ic)
            & (pidx_r_ref[...] == pidx_c_ref[0:1, pl.ds(j * tr, tr)])
            & (ext_r_ref[...] != ext_c_ref[0:1, pl.ds(j * tr, tr)])
        )
        mm = msim_r_ref[...] * msim_c_ref[0:1, pl.ds(j * tr, tr)]
        val = jnp.where(cand, v * mm, 0.0) + inv
        m0 = jnp.max(val)

        @pl.when(m0 >= rvals[K - 1])
        def _extract_and_merge():
            flat = ir * bn + ic
            vv = val
            ff = flat
            tile_v = []
            tile_f = []
            for k in range(K):
                m = m0 if k == 0 else jnp.max(vv)
                bf = jnp.max(jnp.where(vv == m, ff, -1))
                tile_v.append(m)
                tile_f.append(bf)
                vv = jnp.where(ff == bf, NEG_INF, vv)

            vals = [rvals[k] for k in range(K)] + tile_v
            flats = [rflats[k] for k in range(K)] + tile_f
            for slot in range(K):
                bv, bf = vals[0], flats[0]
                for t in range(1, len(vals)):
                    c = (vals[t] > bv) | ((vals[t] == bv) & (flats[t] > bf))
                    bv = jnp.where(c, vals[t], bv)
                    bf = jnp.where(c, flats[t], bf)
                rvals[slot] = bv
                rflats[slot] = bf
                nv, nfl = [], []
                for t in range(len(vals)):
                    hit = flats[t] == bf
                    nv.append(jnp.where(hit, jnp.float32(NEG_INF), vals[t]))
                    nfl.append(jnp.where(hit, jnp.int32(-2), flats[t]))
                vals, flats = nv, nfl

    @pl.when((i == ngrid - 1) & (j == ngrid - 1))
    def _finalize():
        rowscratch[...] = jnp.zeros_like(rowscratch)
        for k in range(K):
            f = rflats[k]
            ik = f // bn
            jk = f % bn
            rowscratch[2 * k:2 * k + 1, :] = as_full_ref[pl.ds(ik, 1), :]
            rowscratch[2 * k + 1:2 * k + 2, :] = as_full_ref[pl.ds(jk, 1), :]
        r = rowscratch[...]
        sr = jnp.where(
            jax.lax.broadcasted_iota(jnp.int32, r.shape, 1) < nf, 1.0, -1.0
        ).astype(jnp.float32)
        dr = jax.lax.dot_general(
            r * sr, as_full_ref[...],
            dimension_numbers=(((1,), (1,)), ((), ())),
            preferred_element_type=jnp.float32,
        )
        total = jnp.sum(jnp.abs(dr))
        out_ref[0, 0] = GAMMA * total / (K * 2 * bn)


def kernel(frozen_embeddings, feature_embeddings, proto_sim, labels):
    b, n, d = frozen_embeddings.shape
    nf = feature_embeddings.shape[2]
    p = proto_sim.shape[1]
    bn = b * n
    tr = 256
    ngrid = (bn + tr - 1) // tr
    bnp = ngrid * tr
    pp = ((p + 127) // 128) * 128
    dd = nf + d

    xf = feature_embeddings.reshape(bn, nf)
    xz = frozen_embeddings.reshape(bn, d)
    ps = jnp.transpose(proto_sim, (0, 2, 1)).reshape(bn, p)

    pad = bnp - bn
    xf = jnp.pad(xf, ((0, pad), (0, 0)))
    xz = jnp.pad(xz, ((0, pad), (0, 0)))
    ps = jnp.pad(ps, ((0, pad), (0, pp - p)), constant_values=-1.0)

    a, a_s, msim, pidx = pl.pallas_call(
        functools.partial(_prep_kernel, nf=nf),
        grid=(ngrid,),
        in_specs=[
            pl.BlockSpec((tr, nf), lambda i: (i, 0)),
            pl.BlockSpec((tr, d), lambda i: (i, 0)),
            pl.BlockSpec((tr, pp), lambda i: (i, 0)),
        ],
        out_specs=[
            pl.BlockSpec((tr, dd), lambda i: (i, 0)),
            pl.BlockSpec((tr, dd), lambda i: (i, 0)),
            pl.BlockSpec((tr, 1), lambda i: (i, 0)),
            pl.BlockSpec((tr, 1), lambda i: (i, 0)),
        ],
        out_shape=[
            jax.ShapeDtypeStruct((bnp, dd), jnp.float32),
            jax.ShapeDtypeStruct((bnp, dd), jnp.float32),
            jax.ShapeDtypeStruct((bnp, 1), jnp.float32),
            jax.ShapeDtypeStruct((bnp, 1), jnp.int32),
        ],
    )(xf, xz, ps)

    ext = jnp.repeat(labels, n).astype(jnp.int32)
    ext = jnp.pad(ext, (0, pad), constant_values=-1)
    ext_r = ext.reshape(bnp, 1)
    ext_c = ext.reshape(1, bnp)
    msim_c = msim.reshape(1, bnp)
    pidx_c = pidx.reshape(1, bnp)

    loss = pl.pallas_call(
        functools.partial(_search_kernel, bn=bn, nf=nf, tr=tr, ngrid=ngrid),
        grid=(ngrid, ngrid),
        in_specs=[
            pl.BlockSpec((tr, dd), lambda i, j: (i, 0)),
            pl.BlockSpec((bnp, dd), lambda i, j: (0, 0)),
            pl.BlockSpec((tr, 1), lambda i, j: (i, 0)),
            pl.BlockSpec((tr, 1), lambda i, j: (i, 0)),
            pl.BlockSpec((tr, 1), lambda i, j: (i, 0)),
            pl.BlockSpec((1, bnp), lambda i, j: (0, 0)),
            pl.BlockSpec((1, bnp), lambda i, j: (0, 0)),
            pl.BlockSpec((1, bnp), lambda i, j: (0, 0)),
        ],
        out_specs=pl.BlockSpec((1, 1), lambda i, j: (0, 0),
                               memory_space=pltpu.SMEM),
        out_shape=jax.ShapeDtypeStruct((1, 1), jnp.float32),
        scratch_shapes=[
            pltpu.SMEM((8,), jnp.float32),
            pltpu.SMEM((8,), jnp.int32),
            pltpu.VMEM((16, dd), jnp.float32),
        ],
    )(a, a_s, msim, pidx, ext_r, msim_c, pidx_c, ext_c)

    return loss[0, 0]


# speedup vs baseline: 1.2642x; 1.2642x over previous
import functools

import jax
import jax.numpy as jnp
from jax.experimental import pallas as pl
from jax.experimental.pallas import tpu as pltpu

K = 5
GAMMA = 1.0
NEG_INF = float("-inf")


def _prep_kernel(xf_ref, xz_ref, ps_ref, a_ref, as_ref, msim_ref, pidx_ref,
                 *, nf):
    xf = xf_ref[...]
    xz = xz_ref[...]
    nrm_f = jnp.maximum(jnp.sqrt(jnp.sum(xf * xf, axis=1, keepdims=True)), 1e-8)
    nrm_z = jnp.maximum(jnp.sqrt(jnp.sum(xz * xz, axis=1, keepdims=True)), 1e-8)
    xnf = xf / nrm_f
    xnz = xz / nrm_z
    a_ref[:, :nf] = xnf
    a_ref[:, nf:] = xnz
    as_ref[:, :nf] = xnf
    as_ref[:, nf:] = -xnz

    ps = ps_ref[...]
    pp = ps.shape[1]
    pmax = jnp.max(ps, axis=1, keepdims=True)
    li = jax.lax.broadcasted_iota(jnp.int32, ps.shape, 1)
    pidx = jnp.min(jnp.where(ps == pmax, li, pp), axis=1, keepdims=True)
    msim_ref[...] = pmax
    pidx_ref[...] = pidx


def _search_kernel(a_rows_ref, as_full_ref, msim_r_ref, pidx_r_ref, ext_r_ref,
                   msim_c_ref, pidx_c_ref, ext_c_ref, out_ref,
                   rvals, rflats, rowscratch, *, bn, nf, tr, cw, ngrid, nh):
    i = pl.program_id(0)
    h = pl.program_id(1)

    @pl.when((i == 0) & (h == 0))
    def _init():
        for k in range(K):
            rvals[k] = jnp.float32(NEG_INF)
            rflats[k] = jnp.int32(-1)

    @pl.when(h * cw + cw - 1 > i * tr)
    def _cell():
        c0 = h * cw
        v = jax.lax.dot_general(
            a_rows_ref[...], as_full_ref[pl.ds(c0, cw), :],
            dimension_numbers=(((1,), (1,)), ((), ())),
            preferred_element_type=jnp.float32,
        )
        ir = jax.lax.broadcasted_iota(jnp.int32, (tr, 1), 0) + i * tr
        ic = jax.lax.broadcasted_iota(jnp.int32, (1, cw), 1) + c0
        cand = (
            (ir < ic)
            & (pidx_r_ref[...] == pidx_c_ref[0:1, pl.ds(c0, cw)])
            & (ext_r_ref[...] != ext_c_ref[0:1, pl.ds(c0, cw)])
        )
        mm = msim_r_ref[...] * msim_c_ref[0:1, pl.ds(c0, cw)]
        val = jnp.where(cand, v * mm, 0.0)
        m0 = jnp.max(val)

        @pl.when(m0 >= rvals[K - 1])
        def _extract_and_merge():
            flat = jnp.where((ir < bn) & (ic < bn), ir * bn + ic, -1)
            vv = val
            ff = flat
            tile_v = []
            tile_f = []
            for k in range(K):
                m = m0 if k == 0 else jnp.max(vv)
                bf = jnp.max(jnp.where(vv == m, ff, -1))
                tile_v.append(m)
                tile_f.append(bf)
                vv = jnp.where(ff == bf, NEG_INF, vv)

            vals = [rvals[k] for k in range(K)] + tile_v
            flats = [rflats[k] for k in range(K)] + tile_f
            for slot in range(K):
                bv, bf = vals[0], flats[0]
                for t in range(1, len(vals)):
                    c = (vals[t] > bv) | ((vals[t] == bv) & (flats[t] > bf))
                    bv = jnp.where(c, vals[t], bv)
                    bf = jnp.where(c, flats[t], bf)
                rvals[slot] = bv
                rflats[slot] = bf
                nv, nfl = [], []
                for t in range(len(vals)):
                    hit = (flats[t] == bf) & (flats[t] >= 0)
                    nv.append(jnp.where(hit, jnp.float32(NEG_INF), vals[t]))
                    nfl.append(jnp.where(hit, jnp.int32(-2), flats[t]))
                vals, flats = nv, nfl

    @pl.when((i == ngrid - 1) & (h == nh - 1))
    def _finalize():
        rowscratch[...] = jnp.zeros_like(rowscratch)
        for k in range(K):
            f = rflats[k]
            ik = f // bn
            jk = f % bn
            rowscratch[2 * k:2 * k + 1, :] = as_full_ref[pl.ds(ik, 1), :]
            rowscratch[2 * k + 1:2 * k + 2, :] = as_full_ref[pl.ds(jk, 1), :]
        r = rowscratch[...]
        sr = jnp.where(
            jax.lax.broadcasted_iota(jnp.int32, r.shape, 1) < nf, 1.0, -1.0
        ).astype(jnp.float32)
        dr = jax.lax.dot_general(
            r * sr, as_full_ref[...],
            dimension_numbers=(((1,), (1,)), ((), ())),
            preferred_element_type=jnp.float32,
        )
        total = jnp.sum(jnp.abs(dr))
        out_ref[0, 0] = GAMMA * total / (K * 2 * bn)


def kernel(frozen_embeddings, feature_embeddings, proto_sim, labels):
    b, n, d = frozen_embeddings.shape
    nf = feature_embeddings.shape[2]
    p = proto_sim.shape[1]
    bn = b * n
    tr = 256
    nh = 2
    ngrid = (bn + tr - 1) // tr
    bnp = ngrid * tr
    cw = bnp // nh
    pp = ((p + 127) // 128) * 128
    dd = nf + d

    xf = feature_embeddings.reshape(bn, nf)
    xz = frozen_embeddings.reshape(bn, d)
    ps = jnp.transpose(proto_sim, (0, 2, 1)).reshape(bn, p)

    pad = bnp - bn
    xf = jnp.pad(xf, ((0, pad), (0, 0)))
    xz = jnp.pad(xz, ((0, pad), (0, 0)))
    ps = jnp.pad(ps, ((0, pad), (0, pp - p)), constant_values=-1.0)

    a, a_s, msim, pidx = pl.pallas_call(
        functools.partial(_prep_kernel, nf=nf),
        grid=(ngrid,),
        in_specs=[
            pl.BlockSpec((tr, nf), lambda i: (i, 0)),
            pl.BlockSpec((tr, d), lambda i: (i, 0)),
            pl.BlockSpec((tr, pp), lambda i: (i, 0)),
        ],
        out_specs=[
            pl.BlockSpec((tr, dd), lambda i: (i, 0)),
            pl.BlockSpec((tr, dd), lambda i: (i, 0)),
            pl.BlockSpec((tr, 1), lambda i: (i, 0)),
            pl.BlockSpec((tr, 1), lambda i: (i, 0)),
        ],
        out_shape=[
            jax.ShapeDtypeStruct((bnp, dd), jnp.float32),
            jax.ShapeDtypeStruct((bnp, dd), jnp.float32),
            jax.ShapeDtypeStruct((bnp, 1), jnp.float32),
            jax.ShapeDtypeStruct((bnp, 1), jnp.int32),
        ],
    )(xf, xz, ps)

    ext = jnp.repeat(labels, n).astype(jnp.int32)
    ext = jnp.pad(ext, (0, pad), constant_values=-1)
    ext_r = ext.reshape(bnp, 1)
    ext_c = ext.reshape(1, bnp)
    msim_c = msim.reshape(1, bnp)
    pidx_c = pidx.reshape(1, bnp)

    loss = pl.pallas_call(
        functools.partial(_search_kernel, bn=bn, nf=nf, tr=tr, cw=cw,
                          ngrid=ngrid, nh=nh),
        grid=(ngrid, nh),
        in_specs=[
            pl.BlockSpec((tr, dd), lambda i, h: (i, 0)),
            pl.BlockSpec((bnp, dd), lambda i, h: (0, 0)),
            pl.BlockSpec((tr, 1), lambda i, h: (i, 0)),
            pl.BlockSpec((tr, 1), lambda i, h: (i, 0)),
            pl.BlockSpec((tr, 1), lambda i, h: (i, 0)),
            pl.BlockSpec((1, bnp), lambda i, h: (0, 0)),
            pl.BlockSpec((1, bnp), lambda i, h: (0, 0)),
            pl.BlockSpec((1, bnp), lambda i, h: (0, 0)),
        ],
        out_specs=pl.BlockSpec((1, 1), lambda i, h: (0, 0),
                               memory_space=pltpu.SMEM),
        out_shape=jax.ShapeDtypeStruct((1, 1), jnp.float32),
        scratch_shapes=[
            pltpu.SMEM((8,), jnp.float32),
            pltpu.SMEM((8,), jnp.int32),
            pltpu.VMEM((16, dd), jnp.float32),
        ],
    )(a, a_s, msim, pidx, ext_r, msim_c, pidx_c, ext_c)

    return loss[0, 0]
